# baseline (speedup 1.0000x reference)
import functools

import jax
import jax.numpy as jnp
from jax import lax
from jax.experimental import pallas as pl
from jax.experimental.pallas import tpu as pltpu
from jax.experimental.pallas import tpu_sc as plsc


@functools.lru_cache(maxsize=None)
def _build_gather(B, V, D):
    info = plsc.get_sparse_core_info()
    NC, NS = info.num_cores, info.num_subcores
    NW = NC * NS
    assert B % NW == 0
    b_per_w = B // NW
    CH = 256
    NBUF = 4
    assert b_per_w % CH == 0
    n_ch = b_per_w // CH
    assert n_ch % NBUF == 0

    mesh = plsc.VectorSubcoreMesh(core_axis_name="c", subcore_axis_name="s")

    @functools.partial(
        pl.kernel,
        mesh=mesh,
        compiler_params=pltpu.CompilerParams(use_tc_tiling_on_sc=False),
        out_type=jax.ShapeDtypeStruct((B, D), jnp.float32),
        scratch_types=(
            [pltpu.VMEM((b_per_w,), jnp.int32),
             pltpu.VMEM((NBUF, CH, D), jnp.float32)]
            + [pltpu.SemaphoreType.DMA] * (2 * NBUF)
        ),
    )
    def gather_kernel(idx_hbm, table_hbm, out_hbm, idx_v, rows_v, *sems):
        gsems, psems = sems[:NBUF], sems[NBUF:]
        wid = lax.axis_index("s") * NC + lax.axis_index("c")
        base = wid * b_per_w
        pltpu.sync_copy(idx_hbm.at[pl.ds(base, b_per_w)], idx_v)

        def start_gather(j, b):
            pltpu.async_copy(
                table_hbm.at[idx_v.at[pl.ds(j * CH, CH)]], rows_v.at[b], gsems[b])

        def wait_gather(b):
            pltpu.make_async_copy(
                table_hbm.at[pl.ds(0, CH)], rows_v.at[b], gsems[b]).wait()

        def start_put(j, b):
            pltpu.async_copy(
                rows_v.at[b], out_hbm.at[pl.ds(base + j * CH, CH)], psems[b])

        def wait_put(b):
            pltpu.make_async_copy(
                rows_v.at[b], out_hbm.at[pl.ds(0, CH)], psems[b]).wait()

        for j in range(NBUF - 1):
            start_gather(j, j)

        def group(g, carry):
            for b in range(NBUF):
                j = g * NBUF + b
                wait_gather(b)
                start_put(j, b)
                gj = j + NBUF - 1
                gb = (b - 1) % NBUF

                @pl.when(gj < n_ch)
                def _():
                    @pl.when(j > 0)
                    def _():
                        wait_put(gb)
                    start_gather(gj, gb)
            return carry

        lax.fori_loop(0, n_ch // NBUF, group, 0)

        for b in range(NBUF):
            wait_put(b)

    return gather_kernel


def kernel(x, table):
    V, D = table.shape
    B = x.size
    xf = x.reshape(-1).astype(jnp.int32)
    out = _build_gather(B, V, D)(xf, table)
    return out.reshape(x.shape + (D,))


# speedup vs baseline: 1.0061x; 1.0061x over previous
import functools

import jax
import jax.numpy as jnp
from jax import lax
from jax.experimental import pallas as pl
from jax.experimental.pallas import tpu as pltpu
from jax.experimental.pallas import tpu_sc as plsc


@functools.lru_cache(maxsize=None)
def _build_gather(N, S, V, D):
    B = N * S
    info = plsc.get_sparse_core_info()
    NC, NS = info.num_cores, info.num_subcores
    NW = NC * NS
    assert N % NW == 0
    r_per_w = N // NW
    b_per_w = r_per_w * S
    CR = 4
    CH = CR * S
    NBUF = 4
    assert r_per_w % CR == 0
    n_ch = r_per_w // CR
    assert n_ch % NBUF == 0

    mesh = plsc.VectorSubcoreMesh(core_axis_name="c", subcore_axis_name="s")

    @functools.partial(
        pl.kernel,
        mesh=mesh,
        compiler_params=pltpu.CompilerParams(use_tc_tiling_on_sc=False),
        out_type=jax.ShapeDtypeStruct((N, S, D), jnp.float32),
        scratch_types=(
            [pltpu.VMEM((b_per_w,), jnp.int32),
             pltpu.VMEM((NBUF, CH, D), jnp.float32)]
            + [pltpu.SemaphoreType.DMA] * (2 * NBUF)
        ),
    )
    def gather_kernel(x_hbm, table_hbm, out_hbm, idx_v, rows_v, *sems):
        gsems, psems = sems[:NBUF], sems[NBUF:]
        wid = lax.axis_index("s") * NC + lax.axis_index("c")
        row0 = wid * r_per_w
        base = wid * b_per_w
        pltpu.sync_copy(x_hbm.at[pl.ds(base, b_per_w)], idx_v)

        def start_gather(j, b):
            pltpu.async_copy(
                table_hbm.at[idx_v.at[pl.ds(j * CH, CH)]], rows_v.at[b], gsems[b])

        def wait_gather(b):
            pltpu.make_async_copy(
                table_hbm.at[pl.ds(0, CH)], rows_v.at[b], gsems[b]).wait()

        def start_put(j, b):
            for r in range(CR):
                pltpu.async_copy(
                    rows_v.at[b].at[pl.ds(r * S, S)],
                    out_hbm.at[row0 + j * CR + r], psems[b])

        def wait_put(b):
            for _ in range(CR):
                pltpu.make_async_copy(
                    rows_v.at[b].at[pl.ds(0, S)], out_hbm.at[0], psems[b]).wait()

        for j in range(NBUF - 1):
            start_gather(j, j)

        def group(g, carry):
            for b in range(NBUF):
                j = g * NBUF + b
                wait_gather(b)
                start_put(j, b)
                gj = j + NBUF - 1
                gb = (b - 1) % NBUF

                @pl.when(gj < n_ch)
                def _():
                    @pl.when(j > 0)
                    def _():
                        wait_put(gb)
                    start_gather(gj, gb)
            return carry

        lax.fori_loop(0, n_ch // NBUF, group, 0)

        for b in range(NBUF):
            wait_put(b)

    return gather_kernel


def kernel(x, table):
    V, D = table.shape
    N, S = x.shape
    xf = x.reshape(-1).astype(jnp.int32)
    return _build_gather(N, S, V, D)(xf, table)
